# baseline (speedup 1.0000x reference)
import functools

import jax
import jax.numpy as jnp
from jax import lax
from jax.experimental import pallas as pl
from jax.experimental.pallas import tpu as pltpu
from jax.experimental.pallas import tpu_sc as plsc

_C = 128
_A = 100000
_B = 3
_M = 8
_T = 2
_L = 16
_NCORES = 2
_NSUB = 16
_NTILES = _NCORES * _NSUB
_AC = 32
_NH = _AC // _L
_NCHUNKS = _A // _AC
_PER_TILE = -(-_NCHUNKS // _NTILES)
_XBLK = _C * _B * _AC
_W32 = _C // 32


def _softmax_body(w_ref, o_ref):
    w = w_ref[...]
    m = jnp.max(w, axis=1, keepdims=True)
    e = jnp.exp(w - m)
    o_ref[...] = e / jnp.sum(e, axis=1, keepdims=True)


def _softmax_tc(W):
    return pl.pallas_call(
        _softmax_body,
        out_shape=jax.ShapeDtypeStruct(W.shape, W.dtype),
    )(W)


def _chunk_idx(ci, si, j):
    tile = ci * _NSUB + si
    return jnp.minimum(tile * _PER_TILE + j, _NCHUNKS - 1)


def _sc_step(v, Xp, mbits, Wf):
    mesh = plsc.VectorSubcoreMesh(core_axis_name="c", subcore_axis_name="s")

    @functools.partial(
        pl.kernel,
        out_type=jax.ShapeDtypeStruct((_A,), jnp.float32),
        mesh=mesh,
        scratch_types=[
            pltpu.VMEM((_A,), jnp.float32),
            pltpu.SMEM((_C * _M,), jnp.float32),
            pltpu.VMEM((_C * _M,), jnp.float32),
        ],
        compiler_params=pltpu.CompilerParams(
            use_tc_tiling_on_sc=False, needs_layout_passes=False),
    )
    def step(v_hbm, x_hbm, mask_hbm, w_hbm, out_hbm, v_scr, w_scr, w_stage):
        pltpu.sync_copy(v_hbm, v_scr)
        pltpu.sync_copy(w_hbm, w_stage)

        def w_to_smem(i, _):
            wvec = w_stage[pl.ds(i * _L, _L)]
            for j in range(_L):
                w_scr[i * _L + j] = wvec[j]
            return 0

        lax.fori_loop(0, (_M * _C) // _L, w_to_smem, 0)

        iota3 = lax.iota(jnp.int32, _L) * _B
        one = jnp.float32(1.0)

        def body(x_buf, m_buf, vin_buf, out_buf):
            accs = [[jnp.zeros((_L,), jnp.float32) for _ in range(_M)]
                    for _ in range(_NH)]
            for w in range(_W32):
                mw = [m_buf[pl.ds(w * _AC + h * _L, _L)] for h in range(_NH)]
                for k in range(32):
                    c = w * 32 + k
                    ws = [w_scr[c * _M + m] for m in range(_M)]
                    for h in range(_NH):
                        base = c * _B * _AC + h * _B * _L
                        xi0 = plsc.load_gather(x_buf, [iota3 + base])
                        xi1 = plsc.load_gather(x_buf, [iota3 + (base + 1)])
                        xi2 = plsc.load_gather(x_buf, [iota3 + (base + 2)])
                        g0 = plsc.load_gather(v_scr, [xi0])
                        g1 = plsc.load_gather(v_scr, [xi1])
                        g2 = plsc.load_gather(v_scr, [xi2])
                        cv = g0 * g1 * g2
                        bit = (mw[h] >> k) & 1
                        cv = jnp.where(bit != 0, one, cv)
                        for m in range(_M):
                            accs[h][m] = accs[h][m] + ws[m] * cv
            for h in range(_NH):
                p = one - accs[h][0]
                for m in range(1, _M):
                    p = p * (one - accs[h][m])
                r = jnp.clip(one - p, 0.0, 1.0)
                vin = vin_buf[pl.ds(h * _L, _L)]
                vnew = one - (one - vin) * (one - r)
                out_buf[pl.ds(h * _L, _L)] = jnp.clip(vnew, 0.0, 1.0)

        pltpu.emit_pipeline(
            body,
            grid=(_NCORES, _NSUB, _PER_TILE),
            in_specs=[
                pl.BlockSpec((_XBLK,),
                             index_map=lambda ci, si, j: (_chunk_idx(ci, si, j),)),
                pl.BlockSpec((_AC * _W32,),
                             index_map=lambda ci, si, j: (_chunk_idx(ci, si, j),)),
                pl.BlockSpec((_AC,),
                             index_map=lambda ci, si, j: (_chunk_idx(ci, si, j),)),
            ],
            out_specs=[
                pl.BlockSpec((_AC,),
                             index_map=lambda ci, si, j: (_chunk_idx(ci, si, j),)),
            ],
            core_axis_name=("c", "s"),
            dimension_semantics=(pltpu.PARALLEL, pltpu.PARALLEL, pltpu.ARBITRARY),
        )(x_hbm, mask_hbm, v_hbm, out_hbm)

    return step(v, Xp, mbits, Wf)


def kernel(v0, X, fact_mask, W):
    Wf = _softmax_tc(W).T.reshape(-1)
    Xp = jnp.transpose(X.reshape(_C, _NCHUNKS, _AC * _B), (1, 0, 2)).reshape(-1)
    shifts = jnp.arange(32, dtype=jnp.uint32)
    mbits = lax.bitcast_convert_type(
        (fact_mask.T.reshape(_A, _W32, 32).astype(jnp.uint32) << shifts)
        .sum(axis=-1, dtype=jnp.uint32),
        jnp.int32)
    mbits = jnp.transpose(mbits.reshape(_NCHUNKS, _AC, _W32), (0, 2, 1)).reshape(-1)
    Xp, mbits, Wf = lax.optimization_barrier((Xp, mbits, Wf))
    v = v0
    for _ in range(_T):
        v = _sc_step(v, Xp, mbits, Wf)
    return v


# speedup vs baseline: 2.1616x; 2.1616x over previous
import functools

import jax
import jax.numpy as jnp
from jax import lax
from jax.experimental import pallas as pl
from jax.experimental.pallas import tpu as pltpu
from jax.experimental.pallas import tpu_sc as plsc

_C = 128
_A = 100000
_B = 3
_M = 8
_T = 2
_L = 16
_NCORES = 2
_NSUB = 16
_NTILES = _NCORES * _NSUB
_AC = 16
_NH = _AC // _L
_NCHUNKS = _A // _AC
_PER_TILE = -(-_NCHUNKS // _NTILES)
_XBLK = _C * _B * _AC
_W32 = _C // 32


def _softmax_body(w_ref, o_ref):
    w = w_ref[...]
    m = jnp.max(w, axis=1, keepdims=True)
    e = jnp.exp(w - m)
    o_ref[...] = e / jnp.sum(e, axis=1, keepdims=True)


def _softmax_tc(W):
    return pl.pallas_call(
        _softmax_body,
        out_shape=jax.ShapeDtypeStruct(W.shape, W.dtype),
    )(W)


def _chunk_idx(ci, si, j):
    tile = ci * _NSUB + si
    return jnp.minimum(tile * _PER_TILE + j, _NCHUNKS - 1)


def _sc_step(v, Xp, mbits, Wf):
    mesh = plsc.VectorSubcoreMesh(core_axis_name="c", subcore_axis_name="s")

    @functools.partial(
        pl.kernel,
        out_type=jax.ShapeDtypeStruct((_A,), jnp.float32),
        mesh=mesh,
        scratch_types=[
            pltpu.VMEM((_A,), jnp.float32),
            pltpu.SMEM((_C * _M,), jnp.float32),
            pltpu.VMEM((_C * _M,), jnp.float32),
        ],
        compiler_params=pltpu.CompilerParams(
            use_tc_tiling_on_sc=False, needs_layout_passes=False),
    )
    def step(v_hbm, x_hbm, mask_hbm, w_hbm, out_hbm, v_scr, w_scr, w_stage):
        pltpu.sync_copy(v_hbm, v_scr)
        pltpu.sync_copy(w_hbm, w_stage)

        def w_to_smem(i, _):
            wvec = w_stage[pl.ds(i * _L, _L)]
            for j in range(_L):
                w_scr[i * _L + j] = wvec[j]
            return 0

        lax.fori_loop(0, (_M * _C) // _L, w_to_smem, 0)

        iota3 = lax.iota(jnp.int32, _L) * _B
        one = jnp.float32(1.0)

        def body(x_buf, m0_buf, m1_buf, m2_buf, m3_buf, vin_buf, out_buf):
            m_bufs = (m0_buf, m1_buf, m2_buf, m3_buf)
            accs = [jnp.zeros((_L,), jnp.float32) for _ in range(_M)]
            for w in range(_W32):
                mw = m_bufs[w][...]
                for k in range(32):
                    c = w * 32 + k
                    base = c * (_B * _L)
                    xi0 = plsc.load_gather(x_buf, [iota3 + base])
                    xi1 = plsc.load_gather(x_buf, [iota3 + (base + 1)])
                    xi2 = plsc.load_gather(x_buf, [iota3 + (base + 2)])
                    g0 = plsc.load_gather(v_scr, [xi0])
                    g1 = plsc.load_gather(v_scr, [xi1])
                    g2 = plsc.load_gather(v_scr, [xi2])
                    cv = g0 * g1 * g2
                    bit = (mw >> k) & 1
                    cv = jnp.where(bit != 0, one, cv)
                    for m in range(_M):
                        accs[m] = accs[m] + w_scr[c * _M + m] * cv
            p = one - accs[0]
            for m in range(1, _M):
                p = p * (one - accs[m])
            r = jnp.clip(one - p, 0.0, 1.0)
            vin = vin_buf[...]
            vnew = one - (one - vin) * (one - r)
            out_buf[...] = jnp.clip(vnew, 0.0, 1.0)

        mask_specs = [
            pl.BlockSpec((_L,),
                         index_map=functools.partial(
                             lambda w, ci, si, j: (w * _NCHUNKS + _chunk_idx(ci, si, j),), w))
            for w in range(_W32)
        ]
        pltpu.emit_pipeline(
            body,
            grid=(_NCORES, _NSUB, _PER_TILE),
            in_specs=[
                pl.BlockSpec((_XBLK,),
                             index_map=lambda ci, si, j: (_chunk_idx(ci, si, j),)),
            ] + mask_specs + [
                pl.BlockSpec((_AC,),
                             index_map=lambda ci, si, j: (_chunk_idx(ci, si, j),)),
            ],
            out_specs=[
                pl.BlockSpec((_AC,),
                             index_map=lambda ci, si, j: (_chunk_idx(ci, si, j),)),
            ],
            core_axis_name=("c", "s"),
            dimension_semantics=(pltpu.PARALLEL, pltpu.PARALLEL, pltpu.ARBITRARY),
        )(x_hbm, mask_hbm, mask_hbm, mask_hbm, mask_hbm, v_hbm, out_hbm)

    return step(v, Xp, mbits, Wf)


def kernel(v0, X, fact_mask, W):
    Wf = _softmax_tc(W).T.reshape(-1)
    Xp = jnp.transpose(X.reshape(_C, _NCHUNKS, _AC * _B), (1, 0, 2)).reshape(-1)
    shifts = jnp.arange(32, dtype=jnp.uint32)
    mbits = lax.bitcast_convert_type(
        (fact_mask.reshape(_W32, 32, _A).astype(jnp.uint32) << shifts[None, :, None])
        .sum(axis=1, dtype=jnp.uint32),
        jnp.int32).reshape(-1)
    Xp, mbits, Wf = lax.optimization_barrier((Xp, mbits, Wf))
    v = v0
    for _ in range(_T):
        v = _sc_step(v, Xp, mbits, Wf)
    return v
